# baseline (speedup 1.0000x reference)
import functools

import jax
import jax.numpy as jnp
from jax import lax
from jax.experimental import pallas as pl
from jax.experimental.pallas import tpu as pltpu
from jax.experimental.pallas import tpu_sc as plsc

NV = 100000
NF = 200000
NC, NS, L = 2, 16, 16
NW = NC * NS
W = 8

FT = 6272
FLAST = NF - 31 * FT
CH = 1568
NCH = FT // CH
NPAD = 100096
VS = NPAD // NS
VN = NPAD // NW
VLAST = NV - 31 * VN


def _sc_scatter_body(vpos_hbm, idx_hbm, zeros_hbm, out_hbm, acc,
                     slab_v, idx0_v, idx1_v, idx2_v,
                     rows0_v, rows1_v, rows2_v, nbuf):
    cid = lax.axis_index("c")
    sid = lax.axis_index("s")
    wid = sid * NC + cid
    idx_refs = (idx0_v, idx1_v, idx2_v)
    row_refs = (rows0_v, rows1_v, rows2_v)

    pltpu.sync_copy(zeros_hbm, acc.at[pl.ds(sid * VS, VS)])

    lanes = lax.iota(jnp.int32, 16)
    zeroi16 = jnp.zeros((16,), jnp.int32)
    zerof16 = jnp.zeros((16,), jnp.float32)

    full_chunks = FLAST // CH

    def stage(ci, n):
        pltpu.sync_copy(
            idx_hbm.at[pl.ds((wid * FT + ci * CH) * 3, n * 3)],
            slab_v.at[pl.ds(0, n * 3)],
        )

        @pl.loop(0, n // 16)
        def _(i):
            fl = (i * 16 + lanes) * 3
            for c in range(3):
                idx_refs[c][pl.ds(ci * CH + i * 16, 16)] = (
                    plsc.load_gather(slab_v, [fl + c])
                )

    @pl.loop(0, full_chunks)
    def _(ci):
        stage(ci, CH)

    @pl.when(wid < NW - 1)
    def _():
        @pl.loop(full_chunks, NCH)
        def _(ci):
            stage(ci, CH)

    @pl.when(wid == NW - 1)
    def _():
        stage(full_chunks, FLAST - full_chunks * CH)

        @pl.loop(FLAST, FT, step=16)
        def _(f):
            for c in range(3):
                idx_refs[c][pl.ds(f, 16)] = zeroi16

    @pl.loop(0, CH * W // 16)
    def _(j):
        flat = j * 16 + lanes
        plsc.store_scatter(nbuf, [flat // W, flat % W], zerof16)

    @pl.loop(0, NCH)
    def _(ci):
        for c in range(3):
            pltpu.sync_copy(
                vpos_hbm.at[idx_refs[c].at[pl.ds(ci * CH, CH)]], row_refs[c]
            )

        @pl.loop(0, CH // 16)
        def _(i):
            col = i * 16 + lanes

            def comp(c, k):
                kk = jnp.full((16,), k, jnp.int32)
                return plsc.load_gather(row_refs[c], [col, kk])

            x0, y0, z0 = comp(0, 0), comp(0, 1), comp(0, 2)
            x1, y1, z1 = comp(1, 0), comp(1, 1), comp(1, 2)
            x2, y2, z2 = comp(2, 0), comp(2, 1), comp(2, 2)
            e1x, e1y, e1z = x1 - x0, y1 - y0, z1 - z0
            e2x, e2y, e2z = x2 - x0, y2 - y0, z2 - z0
            nx = e1y * e2z - e1z * e2y
            ny = e1z * e2x - e1x * e2z
            nz = e1x * e2y - e1y * e2x

            for k, v in ((0, nx), (1, ny), (2, nz)):
                kk = jnp.full((16,), k, jnp.int32)
                plsc.store_scatter(nbuf, [col, kk], v)

        for c in range(3):
            pltpu.sync_copy(
                nbuf, acc.at[idx_refs[c].at[pl.ds(ci * CH, CH)]], add=True
            )

    plsc.subcore_barrier()

    pltpu.sync_copy(
        acc.at[pl.ds(sid * VS, VS)], out_hbm.at[cid, pl.ds(sid * VS, VS)]
    )


@functools.cache
def _sc_scatter():
    return pl.kernel(
        _sc_scatter_body,
        out_type=jax.ShapeDtypeStruct((NC, NPAD, W), jnp.float32),
        mesh=plsc.VectorSubcoreMesh(
            core_axis_name="c", subcore_axis_name="s",
            num_cores=NC, num_subcores=NS,
        ),
        scratch_types=[
            pltpu.VMEM_SHARED((NPAD, W), jnp.float32),
            pltpu.VMEM((CH * 3,), jnp.int32),
            pltpu.VMEM((FT,), jnp.int32),
            pltpu.VMEM((FT,), jnp.int32),
            pltpu.VMEM((FT,), jnp.int32),
            pltpu.VMEM((CH, W), jnp.float32),
            pltpu.VMEM((CH, W), jnp.float32),
            pltpu.VMEM((CH, W), jnp.float32),
            pltpu.VMEM((CH, W), jnp.float32),
        ],
        compiler_params=pltpu.CompilerParams(
            needs_layout_passes=False, use_tc_tiling_on_sc=False
        ),
    )


def _rsqrt(x):
    i = plsc.bitcast(x, jnp.int32)
    i = jnp.int32(0x5F3759DF) - lax.shift_right_logical(i, 1)
    r = plsc.bitcast(i, jnp.float32)
    for _ in range(3):
        r = r * (1.5 - 0.5 * x * r * r)
    return r


def _sc_norm_body(part_hbm, out_hbm, pa, pb, pc):
    cid = lax.axis_index("c")
    sid = lax.axis_index("s")
    wid = sid * NC + cid
    base = wid * VN

    pltpu.sync_copy(part_hbm.at[0, pl.ds(base, VN)], pa)
    pltpu.sync_copy(part_hbm.at[1, pl.ds(base, VN)], pb)

    lanes = lax.iota(jnp.int32, 16)

    @pl.loop(0, (VN + 15) // 16)
    def _(i):
        v = i * 16 + lanes
        m = v < VN

        def comp(k):
            kk = jnp.full((16,), k, jnp.int32)
            return (plsc.load_gather(pa, [v, kk], mask=m)
                    + plsc.load_gather(pb, [v, kk], mask=m))

        sx, sy, sz = comp(0), comp(1), comp(2)
        sq = sx * sx + sy * sy + sz * sz
        ok = sq > 1e-20
        sx = jnp.where(ok, sx, 0.0)
        sy = jnp.where(ok, sy, 0.0)
        sz = jnp.where(ok, sz, 1.0)
        inv = _rsqrt(jnp.where(ok, sq, 1.0))
        v3 = v * 3
        for k, val in ((0, sx * inv), (1, sy * inv), (2, sz * inv)):
            plsc.store_scatter(pc, [v3 + k], val, mask=m)

    @pl.when(wid < NW - 1)
    def _():
        pltpu.sync_copy(pc, out_hbm.at[pl.ds(base * 3, VN * 3)])

    @pl.when(wid == NW - 1)
    def _():
        pltpu.sync_copy(
            pc.at[pl.ds(0, VLAST * 3)],
            out_hbm.at[pl.ds(base * 3, VLAST * 3)],
        )


@functools.cache
def _sc_norm():
    return pl.kernel(
        _sc_norm_body,
        out_type=jax.ShapeDtypeStruct((NV * 3,), jnp.float32),
        mesh=plsc.VectorSubcoreMesh(
            core_axis_name="c", subcore_axis_name="s",
            num_cores=NC, num_subcores=NS,
        ),
        scratch_types=[
            pltpu.VMEM((VN, W), jnp.float32),
            pltpu.VMEM((VN, W), jnp.float32),
            pltpu.VMEM((VN * 3,), jnp.float32),
        ],
        compiler_params=pltpu.CompilerParams(
            needs_layout_passes=False, use_tc_tiling_on_sc=False
        ),
    )


@jax.jit
def kernel(v_pos, t_pos_idx):
    idx_flat = t_pos_idx.astype(jnp.int32).reshape(NF * 3)
    vpos_pad = jnp.pad(v_pos, ((0, NPAD - NV), (0, W - 3)))
    zeros = jnp.zeros((VS, W), jnp.float32)
    partials = _sc_scatter()(vpos_pad, idx_flat, zeros)
    out = _sc_norm()(partials)
    return out.reshape(NV, 3)


# speedup vs baseline: 1.4120x; 1.4120x over previous
import functools

import jax
import jax.numpy as jnp
from jax import lax
from jax.experimental import pallas as pl
from jax.experimental.pallas import tpu as pltpu
from jax.experimental.pallas import tpu_sc as plsc

NV = 100000
NF = 200000
NC, NS, L = 2, 16, 16
NW = NC * NS
W = 8

FT = 6272
CH = 784
NCH = FT // CH
NPAD = 100096
VS = NPAD // NS
VN = NPAD // NW
VLAST = NV - 31 * VN


def _sc_scatter_body(vpos_hbm, idx_hbm, zeros_hbm, out_hbm, acc,
                     idx0_v, idx1_v, idx2_v, rows_v, nbuf_v, gsem, ssem):
    cid = lax.axis_index("c")
    sid = lax.axis_index("s")
    wid = sid * NC + cid
    idx_refs = (idx0_v, idx1_v, idx2_v)

    pltpu.sync_copy(zeros_hbm, acc.at[pl.ds(sid * VS, VS)])

    for c in range(3):
        pltpu.sync_copy(idx_hbm.at[c, pl.ds(wid * FT, FT)], idx_refs[c])

    lanes = lax.iota(jnp.int32, 16)
    zerof16 = jnp.zeros((16,), jnp.float32)

    @pl.loop(0, 2 * CH * W // 16)
    def _(j):
        flat = j * 16 + lanes
        plsc.store_scatter(nbuf_v, [flat // (CH * W), (flat // W) % CH,
                                    flat % W], zerof16)

    def fire_gathers(ci, b):
        for c in range(3):
            pltpu.async_copy(
                vpos_hbm.at[idx_refs[c].at[pl.ds(ci * CH, CH)]],
                rows_v.at[b, c], gsem.at[b],
            )

    def wait_gathers(b):
        for c in range(3):
            pltpu.make_async_copy(
                vpos_hbm.at[idx_refs[c].at[pl.ds(0, CH)]],
                rows_v.at[b, c], gsem.at[b],
            ).wait()

    def fire_scatters(ci, b):
        for c in range(3):
            pltpu.async_copy(
                nbuf_v.at[b],
                acc.at[idx_refs[c].at[pl.ds(ci * CH, CH)]],
                ssem.at[b], add=True,
            )

    def wait_scatters(b):
        for c in range(3):
            pltpu.make_async_copy(
                nbuf_v.at[b],
                acc.at[idx_refs[c].at[pl.ds(0, CH)]],
                ssem.at[b],
            ).wait()

    def compute(b):
        @pl.loop(0, CH // 16)
        def _(i):
            col = i * 16 + lanes

            def comp(c, k):
                cc = jnp.full((16,), c, jnp.int32)
                kk = jnp.full((16,), k, jnp.int32)
                return plsc.load_gather(rows_v, [b * jnp.ones(
                    (16,), jnp.int32), cc, col, kk])

            x0, y0, z0 = comp(0, 0), comp(0, 1), comp(0, 2)
            x1, y1, z1 = comp(1, 0), comp(1, 1), comp(1, 2)
            x2, y2, z2 = comp(2, 0), comp(2, 1), comp(2, 2)
            e1x, e1y, e1z = x1 - x0, y1 - y0, z1 - z0
            e2x, e2y, e2z = x2 - x0, y2 - y0, z2 - z0
            nx = e1y * e2z - e1z * e2y
            ny = e1z * e2x - e1x * e2z
            nz = e1x * e2y - e1y * e2x

            bb = jnp.full((16,), b, jnp.int32)
            for k, v in ((0, nx), (1, ny), (2, nz)):
                kk = jnp.full((16,), k, jnp.int32)
                plsc.store_scatter(nbuf_v, [bb, col, kk], v)

    fire_gathers(0, 0)
    for ci in range(NCH):
        b = ci % 2
        wait_gathers(b)
        if ci + 1 < NCH:
            fire_gathers(ci + 1, 1 - b)
        if ci >= 2:
            wait_scatters(b)
        compute(b)
        fire_scatters(ci, b)
    wait_scatters(NCH % 2)
    wait_scatters(1 - NCH % 2)

    plsc.subcore_barrier()

    pltpu.sync_copy(
        acc.at[pl.ds(sid * VS, VS)], out_hbm.at[cid, pl.ds(sid * VS, VS)]
    )


@functools.cache
def _sc_scatter():
    return pl.kernel(
        _sc_scatter_body,
        out_type=jax.ShapeDtypeStruct((NC, NPAD, W), jnp.float32),
        mesh=plsc.VectorSubcoreMesh(
            core_axis_name="c", subcore_axis_name="s",
            num_cores=NC, num_subcores=NS,
        ),
        scratch_types=[
            pltpu.VMEM_SHARED((NPAD, W), jnp.float32),
            pltpu.VMEM((FT,), jnp.int32),
            pltpu.VMEM((FT,), jnp.int32),
            pltpu.VMEM((FT,), jnp.int32),
            pltpu.VMEM((2, 3, CH, W), jnp.float32),
            pltpu.VMEM((2, CH, W), jnp.float32),
            pltpu.SemaphoreType.DMA((2,)),
            pltpu.SemaphoreType.DMA((2,)),
        ],
        compiler_params=pltpu.CompilerParams(
            needs_layout_passes=False, use_tc_tiling_on_sc=False
        ),
    )


def _rsqrt(x):
    i = plsc.bitcast(x, jnp.int32)
    i = jnp.int32(0x5F3759DF) - lax.shift_right_logical(i, 1)
    r = plsc.bitcast(i, jnp.float32)
    for _ in range(3):
        r = r * (1.5 - 0.5 * x * r * r)
    return r


def _sc_norm_body(part_hbm, out_hbm, pa, pb, pc):
    cid = lax.axis_index("c")
    sid = lax.axis_index("s")
    wid = sid * NC + cid
    base = wid * VN

    pltpu.sync_copy(part_hbm.at[0, pl.ds(base, VN)], pa)
    pltpu.sync_copy(part_hbm.at[1, pl.ds(base, VN)], pb)

    lanes = lax.iota(jnp.int32, 16)

    @pl.loop(0, (VN + 15) // 16)
    def _(i):
        v = i * 16 + lanes
        m = v < VN

        def comp(k):
            kk = jnp.full((16,), k, jnp.int32)
            return (plsc.load_gather(pa, [v, kk], mask=m)
                    + plsc.load_gather(pb, [v, kk], mask=m))

        sx, sy, sz = comp(0), comp(1), comp(2)
        sq = sx * sx + sy * sy + sz * sz
        ok = sq > 1e-20
        sx = jnp.where(ok, sx, 0.0)
        sy = jnp.where(ok, sy, 0.0)
        sz = jnp.where(ok, sz, 1.0)
        inv = _rsqrt(jnp.where(ok, sq, 1.0))
        v3 = v * 3
        for k, val in ((0, sx * inv), (1, sy * inv), (2, sz * inv)):
            f = v3 + k
            plsc.store_scatter(pc, [f // W, f % W], val, mask=m)

    @pl.when(wid < NW - 1)
    def _():
        pltpu.sync_copy(pc, out_hbm.at[pl.ds(base * 3 // W, VN * 3 // W)])

    @pl.when(wid == NW - 1)
    def _():
        pltpu.sync_copy(
            pc.at[pl.ds(0, VLAST * 3 // W)],
            out_hbm.at[pl.ds(base * 3 // W, VLAST * 3 // W)],
        )


@functools.cache
def _sc_norm():
    return pl.kernel(
        _sc_norm_body,
        out_type=jax.ShapeDtypeStruct((NV * 3 // W, W), jnp.float32),
        mesh=plsc.VectorSubcoreMesh(
            core_axis_name="c", subcore_axis_name="s",
            num_cores=NC, num_subcores=NS,
        ),
        scratch_types=[
            pltpu.VMEM((VN, W), jnp.float32),
            pltpu.VMEM((VN, W), jnp.float32),
            pltpu.VMEM((VN * 3 // W, W), jnp.float32),
        ],
        compiler_params=pltpu.CompilerParams(
            needs_layout_passes=False, use_tc_tiling_on_sc=False
        ),
    )


@jax.jit
def kernel(v_pos, t_pos_idx):
    idxT = jnp.pad(t_pos_idx.astype(jnp.int32).T, ((0, 0), (0, NW * FT - NF)))
    vpos_pad = jnp.pad(v_pos, ((0, NPAD - NV), (0, W - 3)))
    zeros = jnp.zeros((VS, W), jnp.float32)
    partials = _sc_scatter()(vpos_pad, idxT, zeros)
    return _sc_norm()(partials).reshape(NV, 3)
